# baseline (speedup 1.0000x reference)
import functools

import jax
import jax.numpy as jnp
from jax.experimental import pallas as pl
from jax.experimental.pallas import tpu as pltpu

_NEG_SLOPE = 0.2
_MASKED = -9e15


def _project_body(x_ref, w_ref, asrc_ref, adst_ref, wh_ref, es_ref, ed_ref):
    wh = jnp.dot(x_ref[...], w_ref[...], preferred_element_type=jnp.float32)
    wh_ref[...] = wh
    es_ref[...] = jnp.dot(wh, asrc_ref[...], preferred_element_type=jnp.float32)
    ed_ref[...] = jnp.dot(wh, adst_ref[...], preferred_element_type=jnp.float32)


def _project(xin, wm, asrc, adst, block_rows):
    n, k = xin.shape
    f = wm.shape[1]
    h = asrc.shape[1]
    grid = (n // block_rows,)
    return pl.pallas_call(
        _project_body,
        grid=grid,
        in_specs=[
            pl.BlockSpec((block_rows, k), lambda i: (i, 0)),
            pl.BlockSpec((k, f), lambda i: (0, 0)),
            pl.BlockSpec((f, h), lambda i: (0, 0)),
            pl.BlockSpec((f, h), lambda i: (0, 0)),
        ],
        out_specs=[
            pl.BlockSpec((block_rows, f), lambda i: (i, 0)),
            pl.BlockSpec((block_rows, h), lambda i: (i, 0)),
            pl.BlockSpec((block_rows, h), lambda i: (i, 0)),
        ],
        out_shape=[
            jax.ShapeDtypeStruct((n, f), jnp.float32),
            jax.ShapeDtypeStruct((n, h), jnp.float32),
            jax.ShapeDtypeStruct((n, h), jnp.float32),
        ],
    )(xin, wm, asrc, adst)


def _flash_body(adj_ref, es_ref, ed_ref, wh_ref, out_ref, m_ref, l_ref, acc_ref,
                *, heads, hid, num_col_blocks, final_mode):
    c = pl.program_id(1)

    @pl.when(c == 0)
    def _init():
        m_ref[...] = jnp.full_like(m_ref, -jnp.inf)
        l_ref[...] = jnp.zeros_like(l_ref)
        acc_ref[...] = jnp.zeros_like(acc_ref)

    mask = adj_ref[...] > 0
    for h in range(heads):
        es = es_ref[:, h : h + 1]
        ed = ed_ref[:, h : h + 1].T
        e = es + ed
        e = jnp.where(e >= 0, e, _NEG_SLOPE * e)
        e = jnp.where(mask, e, _MASKED)
        m_old = m_ref[:, h : h + 1]
        m_new = jnp.maximum(m_old, jnp.max(e, axis=1, keepdims=True))
        p = jnp.exp(e - m_new)
        corr = jnp.exp(m_old - m_new)
        m_ref[:, h : h + 1] = m_new
        l_ref[:, h : h + 1] = l_ref[:, h : h + 1] * corr + jnp.sum(
            p, axis=1, keepdims=True)
        sl = slice(h * hid, (h + 1) * hid)
        acc_ref[:, sl] = acc_ref[:, sl] * corr + jnp.dot(
            p, wh_ref[:, sl], preferred_element_type=jnp.float32)

    @pl.when(c == num_col_blocks - 1)
    def _finalize():
        if final_mode == "elu":
            for h in range(heads):
                sl = slice(h * hid, (h + 1) * hid)
                o = acc_ref[:, sl] / l_ref[:, h : h + 1]
                out_ref[:, sl] = jnp.where(o > 0, o, jnp.exp(o) - 1.0)
        else:
            z = acc_ref[...] / l_ref[...]
            z = jnp.where(z > 0, z, jnp.exp(z) - 1.0)
            z = z - jnp.max(z, axis=1, keepdims=True)
            out_ref[...] = z - jnp.log(
                jnp.sum(jnp.exp(z), axis=1, keepdims=True))


def _flash(adj, es, ed, wh, heads, hid, final_mode, block_rows, block_cols):
    n = adj.shape[0]
    grid = (n // block_rows, n // block_cols)
    body = functools.partial(
        _flash_body,
        heads=heads,
        hid=hid,
        num_col_blocks=grid[1],
        final_mode=final_mode,
    )
    return pl.pallas_call(
        body,
        grid=grid,
        in_specs=[
            pl.BlockSpec((block_rows, block_cols), lambda r, c: (r, c)),
            pl.BlockSpec((block_rows, heads), lambda r, c: (r, 0)),
            pl.BlockSpec((block_cols, heads), lambda r, c: (c, 0)),
            pl.BlockSpec((block_cols, heads * hid), lambda r, c: (c, 0)),
        ],
        out_specs=pl.BlockSpec((block_rows, heads * hid), lambda r, c: (r, 0)),
        out_shape=jax.ShapeDtypeStruct((n, heads * hid), jnp.float32),
        scratch_shapes=[
            pltpu.VMEM((block_rows, heads), jnp.float32),
            pltpu.VMEM((block_rows, heads), jnp.float32),
            pltpu.VMEM((block_rows, heads * hid), jnp.float32),
        ],
        compiler_params=pltpu.CompilerParams(
            dimension_semantics=("arbitrary", "arbitrary"),
        ),
    )(adj, es, ed, wh)


def kernel(x, adj, W0, a0, W1, a1, W2, a2, W3, a3, W_out, a_out):
    n = x.shape[0]
    hid = W0.shape[1]
    heads = 4

    wcat = jnp.concatenate([W0, W1, W2, W3], axis=1)
    asrc = jnp.zeros((heads * hid, heads), jnp.float32)
    adst = jnp.zeros((heads * hid, heads), jnp.float32)
    for i, a in enumerate((a0, a1, a2, a3)):
        asrc = asrc.at[i * hid : (i + 1) * hid, i].set(a[:hid, 0])
        adst = adst.at[i * hid : (i + 1) * hid, i].set(a[hid:, 0])

    block_rows = min(512, n)
    block_cols = min(512, n)

    wh, es, ed = _project(x, wcat, asrc, adst, block_rows)
    h1 = _flash(adj, es, ed, wh, heads, hid, "elu", block_rows, block_cols)

    wh2, es2, ed2 = _project(h1, W_out, a_out[:hid], a_out[hid:], block_rows)
    return _flash(adj, es2, ed2, wh2, 1, hid, "logsoftmax",
                  block_rows, block_cols)
